# baseline (speedup 1.0000x reference)
import jax
import jax.numpy as jnp
from jax.experimental import pallas as pl

_O = 96
_I = 96
_K = 864
_H = 56
_W = 56
_P = _H * _W
_PAD = 64
_HALF = 3.0
_BOUND = 1.5


def _body(x_ref, wf_ref, bias_ref, mask_ref, sel_ref, out_ref):
    w = wf_ref[...]
    step = jnp.max(jnp.abs(w)) / _HALF
    ws = w / step
    col = jax.lax.broadcasted_iota(jnp.int32, (_O, _K), 1)
    g = col & 3
    sum4 = ws + jnp.roll(ws, -1, 1) + jnp.roll(ws, -2, 1) + jnp.roll(ws, -3, 1)
    base = jnp.where(g == 0, sum4, 0.0)
    mean = (base + jnp.roll(base, 1, 1) + jnp.roll(base, 2, 1)
            + jnp.roll(base, 3, 1)) * 0.25
    cent = jnp.round(jnp.clip(mean, -_BOUND, _BOUND))
    dev = jnp.round(jnp.clip(ws - cent, -_BOUND, _BOUND))
    lev = dev + cent
    taps_all = jnp.dot(lev.astype(jnp.bfloat16), sel_ref[...],
                       preferred_element_type=jnp.float32).astype(jnp.bfloat16)

    mL = mask_ref[0:1, :]
    mR = mask_ref[1:2, :]
    zpad = jnp.zeros((_I, _PAD), jnp.bfloat16)

    xb = x_ref[0].astype(jnp.bfloat16)
    xp = jnp.concatenate([zpad, xb, zpad], axis=1)
    acc = jnp.zeros((_O, _P), jnp.float32)
    for t in range(9):
        dh, dw = t // 3 - 1, t % 3 - 1
        s = dh * _W + dw
        xs = xp[:, _PAD + s:_PAD + s + _P]
        if dw == -1:
            xs = xs * mL
        elif dw == 1:
            xs = xs * mR
        acc = acc + jnp.dot(taps_all[:, t * _I:(t + 1) * _I], xs,
                            preferred_element_type=jnp.float32)
    out_ref[0] = acc * step + bias_ref[...]


def kernel(x, weight, bias):
    n = x.shape[0]
    xf = x.reshape(n, _I, _P)
    wf = weight.reshape(_O, _K)
    colp = jnp.arange(_P) % _W
    masks = jnp.stack([(colp >= 1).astype(jnp.bfloat16),
                       (colp <= _W - 2).astype(jnp.bfloat16)])
    kk = jnp.arange(_K)[:, None]
    cc = jnp.arange(_K)[None, :]
    sel = ((cc % _I) * 9 + cc // _I == kk).astype(jnp.bfloat16)
    out = pl.pallas_call(
        _body,
        grid=(n,),
        in_specs=[
            pl.BlockSpec((1, _I, _P), lambda i: (i, 0, 0)),
            pl.BlockSpec((_O, _K), lambda i: (0, 0)),
            pl.BlockSpec((_O, 1), lambda i: (0, 0)),
            pl.BlockSpec((2, _P), lambda i: (0, 0)),
            pl.BlockSpec((_K, _K), lambda i: (0, 0)),
        ],
        out_specs=pl.BlockSpec((1, _O, _P), lambda i: (i, 0, 0)),
        out_shape=jax.ShapeDtypeStruct((n, _O, _P), jnp.float32),
    )(xf, wf, bias.reshape(_O, 1), masks, sel)
    return out.reshape(n, _O, _H, _W)


# speedup vs baseline: 1.1206x; 1.1206x over previous
import jax
import jax.numpy as jnp
from jax.experimental import pallas as pl
from jax.experimental.pallas import tpu as pltpu

_O = 96
_I = 96
_K = 864
_H = 56
_W = 56
_P = _H * _W
_PAD = 64
_PW = _P + 2 * _PAD
_HALF = 3.0
_BOUND = 1.5


def _body(x_ref, wf_ref, bias_ref, mask_ref, sel_ref, out_ref, bbuf):
    w = wf_ref[...]
    step = jnp.max(jnp.abs(w)) / _HALF
    ws = w / step
    col = jax.lax.broadcasted_iota(jnp.int32, (_O, _K), 1)
    g = col & 3
    sum4 = ws + jnp.roll(ws, -1, 1) + jnp.roll(ws, -2, 1) + jnp.roll(ws, -3, 1)
    base = jnp.where(g == 0, sum4, 0.0)
    mean = (base + jnp.roll(base, 1, 1) + jnp.roll(base, 2, 1)
            + jnp.roll(base, 3, 1)) * 0.25
    cent = jnp.round(jnp.clip(mean, -_BOUND, _BOUND))
    dev = jnp.round(jnp.clip(ws - cent, -_BOUND, _BOUND))
    lev = dev + cent
    taps_all = jnp.dot(lev.astype(jnp.bfloat16), sel_ref[...],
                       preferred_element_type=jnp.float32).astype(jnp.bfloat16)

    xb = x_ref[0].astype(jnp.bfloat16)
    zpad = jnp.zeros((_I, _PAD), jnp.bfloat16)
    xp = jnp.concatenate([zpad, xb, zpad], axis=1)
    variants = {
        -1: xp * mask_ref[0:1, :],
        0: xp,
        1: xp * mask_ref[1:2, :],
    }
    for t in range(9):
        dh, dw = t // 3 - 1, t % 3 - 1
        s = dh * _W + dw
        bbuf[t * _I:(t + 1) * _I, :] = variants[dw][:, _PAD + s:_PAD + s + _P]

    acc = jnp.dot(taps_all, bbuf[...], preferred_element_type=jnp.float32)
    out_ref[0] = acc * step + bias_ref[...]


def kernel(x, weight, bias):
    n = x.shape[0]
    xf = x.reshape(n, _I, _P)
    wf = weight.reshape(_O, _K)
    colq = (jnp.arange(_PW) - _PAD) % _W
    masks = jnp.stack([(colq != _W - 1).astype(jnp.bfloat16),
                       (colq != 0).astype(jnp.bfloat16)])
    kk = jnp.arange(_K)[:, None]
    cc = jnp.arange(_K)[None, :]
    sel = ((cc % _I) * 9 + cc // _I == kk).astype(jnp.bfloat16)
    out = pl.pallas_call(
        _body,
        grid=(n,),
        in_specs=[
            pl.BlockSpec((1, _I, _P), lambda i: (i, 0, 0)),
            pl.BlockSpec((_O, _K), lambda i: (0, 0)),
            pl.BlockSpec((_O, 1), lambda i: (0, 0)),
            pl.BlockSpec((2, _PW), lambda i: (0, 0)),
            pl.BlockSpec((_K, _K), lambda i: (0, 0)),
        ],
        out_specs=pl.BlockSpec((1, _O, _P), lambda i: (i, 0, 0)),
        out_shape=jax.ShapeDtypeStruct((n, _O, _P), jnp.float32),
        scratch_shapes=[pltpu.VMEM((_K, _P), jnp.bfloat16)],
    )(xf, wf, bias.reshape(_O, 1), masks, sel)
    return out.reshape(n, _O, _H, _W)


# speedup vs baseline: 1.1753x; 1.0489x over previous
import jax
import jax.numpy as jnp
from jax.experimental import pallas as pl
from jax.experimental.pallas import tpu as pltpu

_O = 96
_I = 96
_K = 864
_H = 56
_W = 56
_P = _H * _W
_PAD = 64
_PW = _P + 2 * _PAD
_HALF = 3.0
_BOUND = 1.5


def _body(x_ref, wf_ref, bias_ref, mask_ref, sel_ref, out_ref, bbuf):
    w = wf_ref[...]
    step = jnp.max(jnp.abs(w)) / _HALF
    ws = w / step
    col = jax.lax.broadcasted_iota(jnp.int32, (_O, _K), 1)
    g = col & 3
    sum4 = ws + jnp.roll(ws, -1, 1) + jnp.roll(ws, -2, 1) + jnp.roll(ws, -3, 1)
    base = jnp.where(g == 0, sum4, 0.0)
    mean = (base + jnp.roll(base, 1, 1) + jnp.roll(base, 2, 1)
            + jnp.roll(base, 3, 1)) * 0.25
    cent = jnp.round(jnp.clip(mean, -_BOUND, _BOUND))
    dev = jnp.round(jnp.clip(ws - cent, -_BOUND, _BOUND))
    lev = dev + cent
    taps_all = jnp.dot(lev.astype(jnp.bfloat16), sel_ref[...],
                       preferred_element_type=jnp.float32).astype(jnp.bfloat16)

    zpad = jnp.zeros((_I, _PAD), jnp.bfloat16)
    for b in range(2):
        xb = x_ref[b].astype(jnp.bfloat16)
        xp = jnp.concatenate([zpad, xb, zpad], axis=1)
        variants = {
            -1: xp * mask_ref[0:1, :],
            0: xp,
            1: xp * mask_ref[1:2, :],
        }
        for t in range(9):
            dh, dw = t // 3 - 1, t % 3 - 1
            s = dh * _W + dw
            bbuf[t * _I:(t + 1) * _I, :] = variants[dw][:, _PAD + s:_PAD + s + _P]

        acc = jnp.dot(taps_all, bbuf[...], preferred_element_type=jnp.float32)
        out_ref[b] = acc * step + bias_ref[...]


def kernel(x, weight, bias):
    n = x.shape[0]
    xf = x.reshape(n, _I, _P)
    wf = weight.reshape(_O, _K)
    colq = (jnp.arange(_PW) - _PAD) % _W
    masks = jnp.stack([(colq != _W - 1).astype(jnp.bfloat16),
                       (colq != 0).astype(jnp.bfloat16)])
    kk = jnp.arange(_K)[:, None]
    cc = jnp.arange(_K)[None, :]
    sel = ((cc % _I) * 9 + cc // _I == kk).astype(jnp.bfloat16)
    out = pl.pallas_call(
        _body,
        grid=(n // 2,),
        in_specs=[
            pl.BlockSpec((2, _I, _P), lambda i: (i, 0, 0)),
            pl.BlockSpec((_O, _K), lambda i: (0, 0)),
            pl.BlockSpec((_O, 1), lambda i: (0, 0)),
            pl.BlockSpec((2, _PW), lambda i: (0, 0)),
            pl.BlockSpec((_K, _K), lambda i: (0, 0)),
        ],
        out_specs=pl.BlockSpec((2, _O, _P), lambda i: (i, 0, 0)),
        out_shape=jax.ShapeDtypeStruct((n, _O, _P), jnp.float32),
        scratch_shapes=[pltpu.VMEM((_K, _P), jnp.bfloat16)],
    )(xf, wf, bias.reshape(_O, 1), masks, sel)
    return out.reshape(n, _O, _H, _W)


# speedup vs baseline: 1.2383x; 1.0536x over previous
import jax
import jax.numpy as jnp
from jax.experimental import pallas as pl
from jax.experimental.pallas import tpu as pltpu

_O = 96
_I = 96
_K = 864
_H = 56
_W = 56
_P = _H * _W
_PAD = 64
_PW = _P + 2 * _PAD
_HALF = 3.0
_BOUND = 1.5
_N = 4


def _body(xf_hbm, wf_ref, bias_ref, mask_ref, sel_ref, o_hbm, xbuf, obuf,
          bbuf, isems, osems):
    for i in range(_N):
        pltpu.make_async_copy(xf_hbm.at[i], xbuf.at[i], isems.at[i]).start()

    w = wf_ref[...]
    step = jnp.max(jnp.abs(w)) / _HALF
    ws = w / step
    col = jax.lax.broadcasted_iota(jnp.int32, (_O, _K), 1)
    g = col & 3
    sum4 = ws + jnp.roll(ws, -1, 1) + jnp.roll(ws, -2, 1) + jnp.roll(ws, -3, 1)
    base = jnp.where(g == 0, sum4, 0.0)
    mean = (base + jnp.roll(base, 1, 1) + jnp.roll(base, 2, 1)
            + jnp.roll(base, 3, 1)) * 0.25
    cent = jnp.round(jnp.clip(mean, -_BOUND, _BOUND))
    dev = jnp.round(jnp.clip(ws - cent, -_BOUND, _BOUND))
    lev = dev + cent
    taps_all = jnp.dot(lev.astype(jnp.bfloat16), sel_ref[...],
                       preferred_element_type=jnp.float32).astype(jnp.bfloat16)

    mL = mask_ref[0:1, :]
    mR = mask_ref[1:2, :]
    zpad = jnp.zeros((_I, _PAD), jnp.bfloat16)
    bias_v = bias_ref[...]

    for i in range(_N):
        pltpu.make_async_copy(xf_hbm.at[i], xbuf.at[i], isems.at[i]).wait()
        xb = xbuf[i].astype(jnp.bfloat16)
        xp = jnp.concatenate([zpad, xb, zpad], axis=1)
        variants = {-1: xp * mL, 0: xp, 1: xp * mR}
        for t in range(9):
            dh, dw = t // 3 - 1, t % 3 - 1
            s = dh * _W + dw
            bbuf[t * _I:(t + 1) * _I, :] = variants[dw][:, _PAD + s:_PAD + s + _P]
        acc = jnp.dot(taps_all, bbuf[...], preferred_element_type=jnp.float32)
        obuf[i] = acc * step + bias_v
        pltpu.make_async_copy(obuf.at[i], o_hbm.at[i], osems.at[i]).start()

    for i in range(_N):
        pltpu.make_async_copy(obuf.at[i], o_hbm.at[i], osems.at[i]).wait()


def kernel(x, weight, bias):
    n = x.shape[0]
    xf = x.reshape(n, _I, _P)
    wf = weight.reshape(_O, _K)
    colq = (jnp.arange(_PW) - _PAD) % _W
    masks = jnp.stack([(colq != _W - 1).astype(jnp.bfloat16),
                       (colq != 0).astype(jnp.bfloat16)])
    kk = jnp.arange(_K)[:, None]
    cc = jnp.arange(_K)[None, :]
    sel = ((cc % _I) * 9 + cc // _I == kk).astype(jnp.bfloat16)
    out = pl.pallas_call(
        _body,
        in_specs=[
            pl.BlockSpec(memory_space=pltpu.MemorySpace.HBM),
            pl.BlockSpec(memory_space=pltpu.MemorySpace.VMEM),
            pl.BlockSpec(memory_space=pltpu.MemorySpace.VMEM),
            pl.BlockSpec(memory_space=pltpu.MemorySpace.VMEM),
            pl.BlockSpec(memory_space=pltpu.MemorySpace.VMEM),
        ],
        out_specs=pl.BlockSpec(memory_space=pltpu.MemorySpace.HBM),
        out_shape=jax.ShapeDtypeStruct((n, _O, _P), jnp.float32),
        scratch_shapes=[
            pltpu.VMEM((_N, _I, _P), jnp.float32),
            pltpu.VMEM((_N, _O, _P), jnp.float32),
            pltpu.VMEM((_K, _P), jnp.bfloat16),
            pltpu.SemaphoreType.DMA((_N,)),
            pltpu.SemaphoreType.DMA((_N,)),
        ],
    )(xf, wf, bias.reshape(_O, 1), masks, sel)
    return out.reshape(n, _O, _H, _W)
